# baseline (speedup 1.0000x reference)
import jax
import jax.numpy as jnp
from jax import lax
from jax.experimental import pallas as pl
from jax.experimental.pallas import tpu as pltpu
from jax.experimental.pallas import tpu_sc as plsc

N = 10000
E = 320000
H = 2
C = 64
HC = H * C
OUT2 = 64 // 3

NCORES = 2
NSUB = 16
NW = NCORES * NSUB
EPT = 10240
NFULL = E // EPT
ETAIL = E - NFULL * EPT

_SLOPE = 0.2


def _leaky(v):
    return jnp.where(v >= 0, v, _SLOPE * v)


MB = 4000


def _mean_body(a_ref, o_ref):
    s = jnp.sum(a_ref[...], axis=0)

    @pl.when(pl.program_id(0) == 0)
    def _():
        o_ref[0, 0] = 0.0
        o_ref[0, 1] = 0.0

    o_ref[0, 0] += s[0]
    o_ref[0, 1] += s[1]


def _attr_sums(edge_attr):
    return pl.pallas_call(
        _mean_body,
        grid=(E // MB,),
        out_shape=jax.ShapeDtypeStruct((1, 2), jnp.float32),
        in_specs=[pl.BlockSpec((MB, 2), lambda i: (i, 0))],
        out_specs=pl.BlockSpec(memory_space=pltpu.SMEM),
    )(edge_attr)


def _aself_body(m_ref, x_ref, ws_ref, ac_ref, o0_ref, o1_ref):
    inv_e = 1.0 / E
    m0 = m_ref[0, 0] * inv_e
    m1 = m_ref[0, 1] * inv_e
    xv = x_ref[...]
    ones = jnp.ones((N, 1), jnp.float32)
    v = jnp.concatenate([xv, ones, m0 * ones, m1 * ones], axis=1)
    pre = jax.lax.dot_general(
        v, ws_ref[...], (((1,), (0,)), ((), ())),
        preferred_element_type=jnp.float32)
    z = _leaky(pre)
    al = jax.lax.dot_general(
        z, ac_ref[...], (((1,), (0,)), ((), ())),
        preferred_element_type=jnp.float32)
    o0_ref[...] = al[:, 0:1]
    o1_ref[...] = al[:, 1:2]


def _aself(msum, x, wsmat, acols):
    return pl.pallas_call(
        _aself_body,
        out_shape=[jax.ShapeDtypeStruct((N, 1), jnp.float32)] * 2,
        in_specs=[
            pl.BlockSpec(memory_space=pltpu.SMEM),
            pl.BlockSpec((N, 1), lambda: (0, 0)),
            pl.BlockSpec((4, HC), lambda: (0, 0)),
            pl.BlockSpec((HC, 2), lambda: (0, 0)),
        ],
        out_specs=[pl.BlockSpec((N, 1), lambda: (0, 0))] * 2,
    )(msum, x, wsmat, acols)


def _gather_body(x_hbm, as0_hbm, as1_hbm, ei_hbm,
                 sj_hbm, si_hbm, g0_hbm, g1_hbm,
                 x_v, as0_v, as1_v, idx_v, o0_v, o1_v, o2_v):
    c = lax.axis_index("c")
    s = lax.axis_index("s")
    wid = s * NCORES + c
    base = wid * EPT
    pltpu.sync_copy(x_hbm, x_v)
    pltpu.sync_copy(as0_hbm, as0_v)
    pltpu.sync_copy(as1_hbm, as1_v)

    def _run(nedge):
        pltpu.sync_copy(ei_hbm.at[0, pl.ds(base, nedge)],
                        idx_v.at[pl.ds(0, nedge)])

        @plsc.parallel_loop(0, nedge // 16, 1, unroll=8)
        def _(i):
            off = pl.multiple_of(i * 16, 16)
            ids = idx_v[pl.ds(off, 16)]
            o0_v[pl.ds(off, 16)] = plsc.load_gather(x_v, [ids])

        pltpu.sync_copy(o0_v.at[pl.ds(0, nedge)],
                        sj_hbm.at[pl.ds(base, nedge)])
        pltpu.sync_copy(ei_hbm.at[1, pl.ds(base, nedge)],
                        idx_v.at[pl.ds(0, nedge)])

        @plsc.parallel_loop(0, nedge // 16, 1, unroll=8)
        def _(i):
            off = pl.multiple_of(i * 16, 16)
            ids = idx_v[pl.ds(off, 16)]
            o0_v[pl.ds(off, 16)] = plsc.load_gather(x_v, [ids])
            o1_v[pl.ds(off, 16)] = plsc.load_gather(as0_v, [ids])
            o2_v[pl.ds(off, 16)] = plsc.load_gather(as1_v, [ids])

        pltpu.sync_copy(o0_v.at[pl.ds(0, nedge)],
                        si_hbm.at[pl.ds(base, nedge)])
        pltpu.sync_copy(o1_v.at[pl.ds(0, nedge)],
                        g0_hbm.at[pl.ds(base, nedge)])
        pltpu.sync_copy(o2_v.at[pl.ds(0, nedge)],
                        g1_hbm.at[pl.ds(base, nedge)])

    @pl.when(wid < NFULL)
    def _():
        _run(EPT)

    @pl.when(wid == NFULL)
    def _():
        _run(ETAIL)


def _gather(x_flat, as0, as1, edge_index):
    mesh = plsc.VectorSubcoreMesh(
        core_axis_name="c", subcore_axis_name="s",
        num_cores=NCORES, num_subcores=NSUB)
    f = pl.kernel(
        _gather_body,
        out_type=tuple(jax.ShapeDtypeStruct((E,), jnp.float32)
                       for _ in range(4)),
        mesh=mesh,
        compiler_params=pltpu.CompilerParams(
            needs_layout_passes=False, use_tc_tiling_on_sc=False),
        scratch_types=[
            pltpu.VMEM((N,), jnp.float32),
            pltpu.VMEM((N,), jnp.float32),
            pltpu.VMEM((N,), jnp.float32),
            pltpu.VMEM((EPT,), jnp.int32),
            pltpu.VMEM((EPT,), jnp.float32),
            pltpu.VMEM((EPT,), jnp.float32),
            pltpu.VMEM((EPT,), jnp.float32),
        ],
    )
    return f(x_flat, as0, as1, edge_index)


BE = 4096


def _edge_body(w_ref, a_ref, sj_ref, si_ref, g0_ref, g1_ref, ea_ref,
               o0_ref, o1_ref):
    sj = sj_ref[...]
    si = si_ref[...]
    ea = ea_ref[...]
    ones = jnp.ones((BE, 1), jnp.float32)
    zeros = jnp.zeros((BE, 3), jnp.float32)
    v = jnp.concatenate([sj, si, ea, ones, zeros], axis=1)
    pre = jax.lax.dot_general(
        v, w_ref[...], (((1,), (0,)), ((), ())),
        preferred_element_type=jnp.float32)
    z = _leaky(pre)
    al = jax.lax.dot_general(
        z, a_ref[...], (((1,), (0,)), ((), ())),
        preferred_element_type=jnp.float32)
    o0_ref[...] = jnp.exp(al[:, 0:1] - g0_ref[...])
    o1_ref[...] = jnp.exp(al[:, 1:2] - g1_ref[...])


def _edge_vals(wmat, acols, sj, si, g0, g1, edge_attr):
    col = pl.BlockSpec((BE, 1), lambda i: (i, 0))
    return pl.pallas_call(
        _edge_body,
        grid=(pl.cdiv(E, BE),),
        out_shape=[jax.ShapeDtypeStruct((E, 1), jnp.float32)] * 2,
        in_specs=[
            pl.BlockSpec((8, HC), lambda i: (0, 0)),
            pl.BlockSpec((HC, 2), lambda i: (0, 0)),
            col, col, col, col,
            pl.BlockSpec((BE, 2), lambda i: (i, 0)),
        ],
        out_specs=[col] * 2,
    )(wmat, acols, sj, si, g0, g1, edge_attr)


def _scatter_body(ei_hbm, p0_hbm, p1_hbm, sj_hbm, out_hbm,
                  idx_v, v0_v, v1_v, sj_v, acc_v):
    c = lax.axis_index("c")
    s = lax.axis_index("s")
    wid = s * NCORES + c
    base = wid * EPT

    @plsc.parallel_loop(0, (4 * N) // 16, 1, unroll=8)
    def _(i):
        off = pl.multiple_of(i * 16, 16)
        acc_v[pl.ds(off, 16)] = jnp.zeros((16,), jnp.float32)

    def _run(nedge):
        pltpu.sync_copy(ei_hbm.at[1, pl.ds(base, nedge)],
                        idx_v.at[pl.ds(0, nedge)])
        pltpu.sync_copy(p0_hbm.at[pl.ds(base, nedge)],
                        v0_v.at[pl.ds(0, nedge)])
        pltpu.sync_copy(p1_hbm.at[pl.ds(base, nedge)],
                        v1_v.at[pl.ds(0, nedge)])
        pltpu.sync_copy(sj_hbm.at[pl.ds(base, nedge)],
                        sj_v.at[pl.ds(0, nedge)])

        @plsc.parallel_loop(0, nedge // 16, 1, unroll=4)
        def _(i):
            off = pl.multiple_of(i * 16, 16)
            ids = idx_v[pl.ds(off, 16)]
            p0 = v0_v[pl.ds(off, 16)]
            p1 = v1_v[pl.ds(off, 16)]
            sjv = sj_v[pl.ds(off, 16)]
            plsc.addupdate_scatter(acc_v, [ids], p0)
            plsc.addupdate_scatter(acc_v, [ids + N], p1)
            plsc.addupdate_scatter(acc_v, [ids + 2 * N], p0 * sjv)
            plsc.addupdate_scatter(acc_v, [ids + 3 * N], p1 * sjv)

    @pl.when(wid < NFULL)
    def _():
        _run(EPT)

    @pl.when(wid == NFULL)
    def _():
        _run(ETAIL)

    pltpu.sync_copy(acc_v, out_hbm.at[wid])


def _scatter(edge_index, p0, p1, sj):
    mesh = plsc.VectorSubcoreMesh(
        core_axis_name="c", subcore_axis_name="s",
        num_cores=NCORES, num_subcores=NSUB)
    f = pl.kernel(
        _scatter_body,
        out_type=jax.ShapeDtypeStruct((NW, 4 * N), jnp.float32),
        mesh=mesh,
        compiler_params=pltpu.CompilerParams(
            needs_layout_passes=False, use_tc_tiling_on_sc=False),
        scratch_types=[
            pltpu.VMEM((EPT,), jnp.int32),
            pltpu.VMEM((EPT,), jnp.float32),
            pltpu.VMEM((EPT,), jnp.float32),
            pltpu.VMEM((EPT,), jnp.float32),
            pltpu.VMEM((4 * N,), jnp.float32),
        ],
    )
    return f(edge_index, p0, p1, sj)


def _final_body(acc_ref, x_ref, u1_ref, cv_ref, o_ref):
    a = acc_ref[0]
    for t in range(1, NW):
        a = a + acc_ref[t]
    den = a[0:2, :] + 1.0
    num = a[2:4, :] + x_ref[...]
    s1 = num / den
    u = u1_ref[...]
    d = jax.lax.dot_general(s1, u, (((0,), (0,)), ((), ())),
                            preferred_element_type=jnp.float32)
    o_ref[...] = d + cv_ref[...]


def _final(acc, xrow, u1, cvec):
    return pl.pallas_call(
        _final_body,
        out_shape=jax.ShapeDtypeStruct((N, OUT2), jnp.float32),
        in_specs=[
            pl.BlockSpec((NW, 4, N), lambda: (0, 0, 0)),
            pl.BlockSpec((1, N), lambda: (0, 0)),
            pl.BlockSpec((2, OUT2), lambda: (0, 0)),
            pl.BlockSpec((1, OUT2), lambda: (0, 0)),
        ],
        out_specs=pl.BlockSpec((N, OUT2), lambda: (0, 0)),
    )(acc, xrow, u1, cvec)


def kernel(x, edge_index, edge_attr, W_l, b_l, W_r, b_r, W_e, att, bias, W2, b2):
    ei = edge_index.astype(jnp.int32)

    wl = W_l[0]
    wr = W_r[0]
    blr = b_l + b_r
    attv = att.reshape(HC)
    z128 = jnp.zeros((HC,), jnp.float32)
    wmat = jnp.stack([wl, wr, W_e[0], W_e[1], blr, z128, z128, z128])
    wsmat = jnp.stack([wl + wr, blr, W_e[0], W_e[1]])
    hmask = (jnp.arange(HC) < C).astype(jnp.float32)
    acols = jnp.stack([attv * hmask, attv * (1.0 - hmask)], axis=1)
    u1 = jnp.einsum("hc,hco->ho", W_l.reshape(H, C), W2.reshape(H, C, OUT2))
    cvec = ((b_l + bias) @ W2 + b2)[None, :]

    msum = _attr_sums(edge_attr)
    as0, as1 = _aself(msum, x, wsmat, acols)
    sj, si, g0, g1 = _gather(x.reshape(N), as0.reshape(N), as1.reshape(N), ei)
    p0, p1 = _edge_vals(wmat, acols, sj[:, None], si[:, None],
                        g0[:, None], g1[:, None], edge_attr)
    acc = _scatter(ei, p0.reshape(E), p1.reshape(E), sj)
    d = _final(acc.reshape(NW, 4, N), x.reshape(1, N), u1, cvec)
    return d.reshape(1, N * OUT2)
